# Initial kernel scaffold; baseline (speedup 1.0000x reference)
import jax
import jax.numpy as jnp
from jax.experimental import pallas as pl


def kernel(token_ids, numeric, table, W1, b1, W2, b2, Wo1, bo1, Wo2, bo2):
    raise NotImplementedError("write your pallas kernel here")


# speedup vs baseline: 2.5856x; 2.5856x over previous
import functools

import jax
import jax.numpy as jnp
from jax import lax
from jax.experimental import pallas as pl
from jax.experimental.pallas import tpu as pltpu
from jax.experimental.pallas import tpu_sc as plsc

B = 16384
T = 12
E = 32
NF = 28
H = 64

NC = 2
NS = 16
NW = NC * NS
APW = B // NW

CA = 64
ROWS = CA * T
IDX_W = 128
NGATHER = ROWS // IDX_W
NCHUNK = APW // CA


def _sc_embed_bag(idx_flat, table):
    mesh = plsc.VectorSubcoreMesh(core_axis_name="c", subcore_axis_name="s")

    @functools.partial(
        pl.kernel,
        mesh=mesh,
        out_type=jax.ShapeDtypeStruct((B, E), jnp.float32),
        scratch_types=[
            pltpu.VMEM((ROWS,), jnp.int32),
            pltpu.VMEM((ROWS, E), jnp.float32),
            pltpu.VMEM((CA, E), jnp.float32),
            pltpu.SemaphoreType.DMA,
        ],
        compiler_params=pltpu.CompilerParams(use_tc_tiling_on_sc=False),
    )
    def body(idx_hbm, table_hbm, out_hbm, idx_v, rows_v, pooled_v, sem):
        wid = lax.axis_index("s") * NC + lax.axis_index("c")

        def chunk_body(c, carry):
            base = wid * APW * T + c * ROWS
            pltpu.sync_copy(idx_hbm.at[pl.ds(base, ROWS)], idx_v)
            handles = [
                pltpu.async_copy(
                    table_hbm.at[idx_v.at[pl.ds(j * IDX_W, IDX_W)]],
                    rows_v.at[pl.ds(j * IDX_W, IDX_W)],
                    sem,
                )
                for j in range(NGATHER)
            ]
            for h in handles:
                h.wait()

            def action_body(a, inner):
                r = a * T
                acc0 = rows_v[r, pl.ds(0, 16)]
                acc1 = rows_v[r, pl.ds(16, 16)]
                for t in range(1, T):
                    acc0 = acc0 + rows_v[r + t, pl.ds(0, 16)]
                    acc1 = acc1 + rows_v[r + t, pl.ds(16, 16)]
                pooled_v[a, pl.ds(0, 16)] = acc0 * (1.0 / T)
                pooled_v[a, pl.ds(16, 16)] = acc1 * (1.0 / T)
                return inner

            lax.fori_loop(0, CA, action_body, 0)
            pltpu.sync_copy(pooled_v, out_hbm.at[pl.ds(wid * APW + c * CA, CA)])
            return carry

        lax.fori_loop(0, NCHUNK, chunk_body, 0)

    return body(idx_flat, table)


def _tc_mlp(token_embed, numeric, W1, b1, W2, b2, Wo1a, Wo1b, bo1, Wo2, bo2):
    BM = 2048

    def body(te_ref, num_ref, W1_ref, b1_ref, W2_ref, b2_ref,
             Wo1a_ref, Wo1b_ref, bo1_ref, Wo2_ref, bo2_ref, out_ref):
        h = jnp.maximum(
            jnp.dot(num_ref[...], W1_ref[...],
                    preferred_element_type=jnp.float32) + b1_ref[...], 0.0)
        ne = jnp.dot(h, W2_ref[...],
                     preferred_element_type=jnp.float32) + b2_ref[...]
        o = jnp.maximum(
            jnp.dot(te_ref[...], Wo1a_ref[...],
                    preferred_element_type=jnp.float32)
            + jnp.dot(ne, Wo1b_ref[...], preferred_element_type=jnp.float32)
            + bo1_ref[...], 0.0)
        out_ref[...] = jnp.dot(o, Wo2_ref[...],
                               preferred_element_type=jnp.float32) + bo2_ref[...]

    full = lambda shape: pl.BlockSpec(shape, lambda i: (0, 0))
    return pl.pallas_call(
        body,
        grid=(B // BM,),
        in_specs=[
            pl.BlockSpec((BM, E), lambda i: (i, 0)),
            pl.BlockSpec((BM, NF), lambda i: (i, 0)),
            full((NF, H)), full((1, H)), full((H, E)), full((1, E)),
            full((E, E)), full((E, E)), full((1, E)),
            full((E, E)), full((1, E)),
        ],
        out_specs=pl.BlockSpec((BM, E), lambda i: (i, 0)),
        out_shape=jax.ShapeDtypeStruct((B, E), jnp.float32),
    )(token_embed, numeric, W1, b1, W2, b2, Wo1a, Wo1b, bo1, Wo2, bo2)


def kernel(token_ids, numeric, table, W1, b1, W2, b2, Wo1, bo1, Wo2, bo2):
    idx_flat = token_ids.astype(jnp.int32).reshape(B * T)
    token_embed = _sc_embed_bag(idx_flat, table)
    return _tc_mlp(
        token_embed, numeric,
        W1, b1.reshape(1, H), W2, b2.reshape(1, E),
        Wo1[:E], Wo1[E:], bo1.reshape(1, E),
        Wo2, bo2.reshape(1, E),
    )


# speedup vs baseline: 2.7609x; 1.0678x over previous
import functools

import jax
import jax.numpy as jnp
from jax import lax
from jax.experimental import pallas as pl
from jax.experimental.pallas import tpu as pltpu
from jax.experimental.pallas import tpu_sc as plsc

B = 16384
T = 12
E = 32
NF = 28
H = 64

NC = 2
NS = 16
NW = NC * NS
APW = B // NW

CA = 64
ROWS = CA * T
IDX_W = 128
NGATHER = ROWS // IDX_W
NCHUNK = APW // CA


def _sc_embed_bag(idx_flat, table):
    mesh = plsc.VectorSubcoreMesh(core_axis_name="c", subcore_axis_name="s")

    @functools.partial(
        pl.kernel,
        mesh=mesh,
        out_type=jax.ShapeDtypeStruct((B, E), jnp.float32),
        scratch_types=[
            pltpu.VMEM((2, ROWS), jnp.int32),
            pltpu.VMEM((2 * ROWS, E), jnp.float32),
            pltpu.VMEM((2, CA, E), jnp.float32),
            pltpu.SemaphoreType.DMA,
            pltpu.SemaphoreType.DMA,
        ],
        compiler_params=pltpu.CompilerParams(use_tc_tiling_on_sc=False),
    )
    def body(idx_hbm, table_hbm, out_hbm, idx_v, rows_v, pooled_v, sem, osem):
        wid = lax.axis_index("s") * NC + lax.axis_index("c")
        idx_base = wid * APW * T

        def fire(c, buf):
            pltpu.sync_copy(idx_hbm.at[pl.ds(idx_base + c * ROWS, ROWS)],
                            idx_v.at[buf])
            return [
                pltpu.async_copy(
                    table_hbm.at[idx_v.at[buf, pl.ds(j * IDX_W, IDX_W)]],
                    rows_v.at[pl.ds(buf * ROWS + j * IDX_W, IDX_W)],
                    sem,
                )
                for j in range(NGATHER)
            ]

        def compute(buf):
            r0 = buf * ROWS

            def action_body(a, inner):
                for u in range(4):
                    r = r0 + (a * 4 + u) * T
                    acc0 = rows_v[r, pl.ds(0, 16)]
                    acc1 = rows_v[r, pl.ds(16, 16)]
                    for t in range(1, T):
                        acc0 = acc0 + rows_v[r + t, pl.ds(0, 16)]
                        acc1 = acc1 + rows_v[r + t, pl.ds(16, 16)]
                    pooled_v[buf, a * 4 + u, pl.ds(0, 16)] = acc0 * (1.0 / T)
                    pooled_v[buf, a * 4 + u, pl.ds(16, 16)] = acc1 * (1.0 / T)
                return inner

            lax.fori_loop(0, CA // 4, action_body, 0)

        gath = fire(0, 0)
        out_handles = [None, None]
        for c in range(NCHUNK):
            nxt = fire(c + 1, (c + 1) % 2) if c + 1 < NCHUNK else []
            for h in gath:
                h.wait()
            if out_handles[c % 2] is not None:
                out_handles[c % 2].wait()
            compute(c % 2)
            out_handles[c % 2] = pltpu.async_copy(
                pooled_v.at[c % 2],
                out_hbm.at[pl.ds(wid * APW + c * CA, CA)],
                osem,
            )
            gath = nxt
        for h in out_handles:
            if h is not None:
                h.wait()

    return body(idx_flat, table)


def _tc_mlp(token_embed, numeric, W1, b1, W2, b2, Wo1a, Wo1b, bo1, Wo2, bo2):
    BM = 2048

    def body(te_ref, num_ref, W1_ref, b1_ref, W2_ref, b2_ref,
             Wo1a_ref, Wo1b_ref, bo1_ref, Wo2_ref, bo2_ref, out_ref):
        h = jnp.maximum(
            jnp.dot(num_ref[...], W1_ref[...],
                    preferred_element_type=jnp.float32) + b1_ref[...], 0.0)
        ne = jnp.dot(h, W2_ref[...],
                     preferred_element_type=jnp.float32) + b2_ref[...]
        o = jnp.maximum(
            jnp.dot(te_ref[...], Wo1a_ref[...],
                    preferred_element_type=jnp.float32)
            + jnp.dot(ne, Wo1b_ref[...], preferred_element_type=jnp.float32)
            + bo1_ref[...], 0.0)
        out_ref[...] = jnp.dot(o, Wo2_ref[...],
                               preferred_element_type=jnp.float32) + bo2_ref[...]

    full = lambda shape: pl.BlockSpec(shape, lambda i: (0, 0))
    return pl.pallas_call(
        body,
        grid=(B // BM,),
        in_specs=[
            pl.BlockSpec((BM, E), lambda i: (i, 0)),
            pl.BlockSpec((BM, NF), lambda i: (i, 0)),
            full((NF, H)), full((1, H)), full((H, E)), full((1, E)),
            full((E, E)), full((E, E)), full((1, E)),
            full((E, E)), full((1, E)),
        ],
        out_specs=pl.BlockSpec((BM, E), lambda i: (i, 0)),
        out_shape=jax.ShapeDtypeStruct((B, E), jnp.float32),
    )(token_embed, numeric, W1, b1, W2, b2, Wo1a, Wo1b, bo1, Wo2, bo2)


def kernel(token_ids, numeric, table, W1, b1, W2, b2, Wo1, bo1, Wo2, bo2):
    idx_flat = token_ids.astype(jnp.int32).reshape(B * T)
    token_embed = _sc_embed_bag(idx_flat, table)
    return _tc_mlp(
        token_embed, numeric,
        W1, b1.reshape(1, H), W2, b2.reshape(1, E),
        Wo1[:E], Wo1[E:], bo1.reshape(1, E),
        Wo2, bo2.reshape(1, E),
    )


# speedup vs baseline: 2.8338x; 1.0264x over previous
import functools

import jax
import jax.numpy as jnp
from jax import lax
from jax.experimental import pallas as pl
from jax.experimental.pallas import tpu as pltpu
from jax.experimental.pallas import tpu_sc as plsc

B = 16384
T = 12
E = 32
NF = 28
H = 64

NC = 2
NS = 16
NW = NC * NS
APW = B // NW

CA = 128
ROWS = CA * T
IDX_W = 128
NGATHER = ROWS // IDX_W
NCHUNK = APW // CA


def _sc_embed_bag(idx_flat, table):
    mesh = plsc.VectorSubcoreMesh(core_axis_name="c", subcore_axis_name="s")

    @functools.partial(
        pl.kernel,
        mesh=mesh,
        out_type=jax.ShapeDtypeStruct((B, E), jnp.float32),
        scratch_types=[
            pltpu.VMEM((2 * ROWS,), jnp.int32),
            pltpu.VMEM((2 * ROWS, E), jnp.float32),
            pltpu.VMEM((2, CA, E), jnp.float32),
            pltpu.SemaphoreType.DMA,
            pltpu.SemaphoreType.DMA,
        ],
        compiler_params=pltpu.CompilerParams(use_tc_tiling_on_sc=False),
    )
    def body(idx_hbm, table_hbm, out_hbm, idx_v, rows_v, pooled_v, sem, osem):
        wid = lax.axis_index("s") * NC + lax.axis_index("c")
        idx_base = wid * APW * T

        def fire(c, buf):
            pltpu.sync_copy(idx_hbm.at[pl.ds(idx_base + c * ROWS, ROWS)],
                            idx_v.at[pl.ds(buf * ROWS, ROWS)])
            return [
                pltpu.async_copy(
                    table_hbm.at[idx_v.at[pl.ds(buf * ROWS + j * IDX_W, IDX_W)]],
                    rows_v.at[pl.ds(buf * ROWS + j * IDX_W, IDX_W)],
                    sem,
                )
                for j in range(NGATHER)
            ]

        def compute(buf):
            r0 = buf * ROWS

            def action_body(a, inner):
                for u in range(4):
                    r = r0 + (a * 4 + u) * T
                    acc0 = rows_v[r, pl.ds(0, 16)]
                    acc1 = rows_v[r, pl.ds(16, 16)]
                    for t in range(1, T):
                        acc0 = acc0 + rows_v[r + t, pl.ds(0, 16)]
                        acc1 = acc1 + rows_v[r + t, pl.ds(16, 16)]
                    pooled_v[buf, a * 4 + u, pl.ds(0, 16)] = acc0 * (1.0 / T)
                    pooled_v[buf, a * 4 + u, pl.ds(16, 16)] = acc1 * (1.0 / T)
                return inner

            lax.fori_loop(0, CA // 4, action_body, 0)

        gath = fire(0, 0)
        out_handles = [None, None]
        for c in range(NCHUNK):
            nxt = fire(c + 1, (c + 1) % 2) if c + 1 < NCHUNK else []
            for h in gath:
                h.wait()
            if out_handles[c % 2] is not None:
                out_handles[c % 2].wait()
            compute(c % 2)
            out_handles[c % 2] = pltpu.async_copy(
                pooled_v.at[c % 2],
                out_hbm.at[pl.ds(wid * APW + c * CA, CA)],
                osem,
            )
            gath = nxt
        for h in out_handles:
            if h is not None:
                h.wait()

    return body(idx_flat, table)


_BM = 2048
_full = lambda shape: pl.BlockSpec(shape, lambda i: tuple(0 for _ in shape))


def _tc_numeric(numeric, W1, b1, W2, b2, Wo1b, bo1):
    def body(num_ref, W1_ref, b1_ref, W2_ref, b2_ref, Wo1b_ref, bo1_ref,
             out_ref):
        h = jnp.maximum(
            jnp.dot(num_ref[...], W1_ref[...],
                    preferred_element_type=jnp.float32) + b1_ref[...], 0.0)
        ne = jnp.dot(h, W2_ref[...],
                     preferred_element_type=jnp.float32) + b2_ref[...]
        out_ref[...] = jnp.dot(ne, Wo1b_ref[...],
                               preferred_element_type=jnp.float32) + bo1_ref[...]

    return pl.pallas_call(
        body,
        grid=(B // _BM,),
        in_specs=[
            pl.BlockSpec((_BM, NF), lambda i: (i, 0)),
            _full((NF, H)), _full((1, H)), _full((H, E)), _full((1, E)),
            _full((E, E)), _full((1, E)),
        ],
        out_specs=pl.BlockSpec((_BM, E), lambda i: (i, 0)),
        out_shape=jax.ShapeDtypeStruct((B, E), jnp.float32),
    )(numeric, W1, b1, W2, b2, Wo1b, bo1)


def _tc_head(token_embed, pre, Wo1a, Wo2, bo2):
    def body(te_ref, pre_ref, Wo1a_ref, Wo2_ref, bo2_ref, out_ref):
        o = jnp.maximum(
            jnp.dot(te_ref[...], Wo1a_ref[...],
                    preferred_element_type=jnp.float32) + pre_ref[...], 0.0)
        out_ref[...] = jnp.dot(o, Wo2_ref[...],
                               preferred_element_type=jnp.float32) + bo2_ref[...]

    return pl.pallas_call(
        body,
        grid=(B // _BM,),
        in_specs=[
            pl.BlockSpec((_BM, E), lambda i: (i, 0)),
            pl.BlockSpec((_BM, E), lambda i: (i, 0)),
            _full((E, E)), _full((E, E)), _full((1, E)),
        ],
        out_specs=pl.BlockSpec((_BM, E), lambda i: (i, 0)),
        out_shape=jax.ShapeDtypeStruct((B, E), jnp.float32),
    )(token_embed, pre, Wo1a, Wo2, bo2)


def kernel(token_ids, numeric, table, W1, b1, W2, b2, Wo1, bo1, Wo2, bo2):
    idx_flat = token_ids.astype(jnp.int32).reshape(B * T)
    token_embed = _sc_embed_bag(idx_flat, table)
    pre = _tc_numeric(numeric, W1, b1.reshape(1, H), W2, b2.reshape(1, E),
                      Wo1[E:], bo1.reshape(1, E))
    return _tc_head(token_embed, pre, Wo1[:E], Wo2, bo2.reshape(1, E))


# speedup vs baseline: 2.8394x; 1.0020x over previous
import functools

import jax
import jax.numpy as jnp
from jax import lax
from jax.experimental import pallas as pl
from jax.experimental.pallas import tpu as pltpu
from jax.experimental.pallas import tpu_sc as plsc

B = 16384
T = 12
E = 32
NF = 28
H = 64

NC = 2
NS = 16
NW = NC * NS
APW = B // NW

CA = 128
ROWS = CA * T
IDX_W = 128
NGATHER = ROWS // IDX_W
NCHUNK = APW // CA


def _sc_embed_bag(idx_flat, table):
    mesh = plsc.VectorSubcoreMesh(core_axis_name="c", subcore_axis_name="s")

    @functools.partial(
        pl.kernel,
        mesh=mesh,
        out_type=jax.ShapeDtypeStruct((B, E), jnp.float32),
        scratch_types=[
            pltpu.VMEM((2, NGATHER, IDX_W), jnp.int32),
            pltpu.VMEM((2 * ROWS, E), jnp.float32),
            pltpu.VMEM((2, CA, E), jnp.float32),
            pltpu.SemaphoreType.DMA,
            pltpu.SemaphoreType.DMA,
        ],
        compiler_params=pltpu.CompilerParams(use_tc_tiling_on_sc=False),
    )
    def body(idx_hbm, table_hbm, out_hbm, idx_v, rows_v, pooled_v, sem, osem):
        wid = lax.axis_index("s") * NC + lax.axis_index("c")
        idx_row_base = wid * (APW * T // IDX_W)

        def fire(c, buf):
            pltpu.sync_copy(
                idx_hbm.at[pl.ds(idx_row_base + c * NGATHER, NGATHER)],
                idx_v.at[buf])
            return [
                pltpu.async_copy(
                    table_hbm.at[idx_v.at[buf, j]],
                    rows_v.at[pl.ds(buf * ROWS + j * IDX_W, IDX_W)],
                    sem,
                )
                for j in range(NGATHER)
            ]

        def compute(buf):
            r0 = buf * ROWS

            def action_body(a, inner):
                for u in range(4):
                    r = r0 + (a * 4 + u) * T
                    acc0 = rows_v[r, pl.ds(0, 16)]
                    acc1 = rows_v[r, pl.ds(16, 16)]
                    for t in range(1, T):
                        acc0 = acc0 + rows_v[r + t, pl.ds(0, 16)]
                        acc1 = acc1 + rows_v[r + t, pl.ds(16, 16)]
                    pooled_v[buf, a * 4 + u, pl.ds(0, 16)] = acc0 * (1.0 / T)
                    pooled_v[buf, a * 4 + u, pl.ds(16, 16)] = acc1 * (1.0 / T)
                return inner

            lax.fori_loop(0, CA // 4, action_body, 0)

        gath = fire(0, 0)
        out_handles = [None, None]
        for c in range(NCHUNK):
            nxt = fire(c + 1, (c + 1) % 2) if c + 1 < NCHUNK else []
            for h in gath:
                h.wait()
            if out_handles[c % 2] is not None:
                out_handles[c % 2].wait()
            compute(c % 2)
            out_handles[c % 2] = pltpu.async_copy(
                pooled_v.at[c % 2],
                out_hbm.at[pl.ds(wid * APW + c * CA, CA)],
                osem,
            )
            gath = nxt
        for h in out_handles:
            if h is not None:
                h.wait()

    return body(idx_flat, table)


_BM = 2048
_full = lambda shape: pl.BlockSpec(shape, lambda i: tuple(0 for _ in shape))


def _tc_numeric(numeric, W1, b1, W2, b2, Wo1b, bo1):
    def body(num_ref, W1_ref, b1_ref, W2_ref, b2_ref, Wo1b_ref, bo1_ref,
             out_ref):
        h = jnp.maximum(
            jnp.dot(num_ref[...], W1_ref[...],
                    preferred_element_type=jnp.float32) + b1_ref[...], 0.0)
        ne = jnp.dot(h, W2_ref[...],
                     preferred_element_type=jnp.float32) + b2_ref[...]
        out_ref[...] = jnp.dot(ne, Wo1b_ref[...],
                               preferred_element_type=jnp.float32) + bo1_ref[...]

    return pl.pallas_call(
        body,
        grid=(B // _BM,),
        in_specs=[
            pl.BlockSpec((_BM, NF), lambda i: (i, 0)),
            _full((NF, H)), _full((1, H)), _full((H, E)), _full((1, E)),
            _full((E, E)), _full((1, E)),
        ],
        out_specs=pl.BlockSpec((_BM, E), lambda i: (i, 0)),
        out_shape=jax.ShapeDtypeStruct((B, E), jnp.float32),
    )(numeric, W1, b1, W2, b2, Wo1b, bo1)


def _tc_head(token_embed, pre, Wo1a, Wo2, bo2):
    def body(te_ref, pre_ref, Wo1a_ref, Wo2_ref, bo2_ref, out_ref):
        o = jnp.maximum(
            jnp.dot(te_ref[...], Wo1a_ref[...],
                    preferred_element_type=jnp.float32) + pre_ref[...], 0.0)
        out_ref[...] = jnp.dot(o, Wo2_ref[...],
                               preferred_element_type=jnp.float32) + bo2_ref[...]

    return pl.pallas_call(
        body,
        grid=(B // _BM,),
        in_specs=[
            pl.BlockSpec((_BM, E), lambda i: (i, 0)),
            pl.BlockSpec((_BM, E), lambda i: (i, 0)),
            _full((E, E)), _full((E, E)), _full((1, E)),
        ],
        out_specs=pl.BlockSpec((_BM, E), lambda i: (i, 0)),
        out_shape=jax.ShapeDtypeStruct((B, E), jnp.float32),
    )(token_embed, pre, Wo1a, Wo2, bo2)


def kernel(token_ids, numeric, table, W1, b1, W2, b2, Wo1, bo1, Wo2, bo2):
    idx2d = token_ids.astype(jnp.int32).reshape(B * T // IDX_W, IDX_W)
    token_embed = _sc_embed_bag(idx2d, table)
    pre = _tc_numeric(numeric, W1, b1.reshape(1, H), W2, b2.reshape(1, E),
                      Wo1[E:], bo1.reshape(1, E))
    return _tc_head(token_embed, pre, Wo1[:E], Wo2, bo2.reshape(1, E))
